# baseline (speedup 1.0000x reference)
import functools

import jax
import jax.numpy as jnp
from jax import lax
from jax.experimental import pallas as pl
from jax.experimental.pallas import tpu as pltpu
from jax.experimental.pallas import tpu_sc as plsc

BATCH = 16384
D = 32
PACK = 4
WIDE = D * PACK
NC = 2
NS = 16
NW = NC * NS
BPW = BATCH // NW
CH = 128
NCHUNK = BPW // CH

_mesh = plsc.VectorSubcoreMesh(core_axis_name="c", subcore_axis_name="s")


@functools.partial(
    pl.kernel,
    out_type=jax.ShapeDtypeStruct((BATCH,), jnp.float32),
    mesh=_mesh,
    scratch_types=[
        pltpu.VMEM((BPW,), jnp.int32),
        pltpu.VMEM((BPW,), jnp.int32),
        pltpu.VMEM((BPW,), jnp.int32),
        pltpu.VMEM((BPW,), jnp.int32),
        pltpu.VMEM((BPW,), jnp.int32),
        pltpu.VMEM((BPW,), jnp.int32),
        pltpu.VMEM((CH, WIDE), jnp.float32),
        pltpu.VMEM((CH, WIDE), jnp.float32),
        pltpu.VMEM((48,), jnp.float32),
        pltpu.VMEM((BPW,), jnp.float32),
        pltpu.SemaphoreType.DMA,
    ],
    compiler_params=pltpu.CompilerParams(
        needs_layout_passes=False, use_tc_tiling_on_sc=False),
)
def _gmf_sc(uid_hbm, iid_hbm, ut_hbm, it_hbm, wb_hbm, out_hbm,
            uidx, iidx, uwid, iwid, ublk, iblk, uwide, iwide, wv, outv, sem):
    wid = lax.axis_index("s") * NC + lax.axis_index("c")
    base = wid * BPW

    pltpu.sync_copy(uid_hbm.at[pl.ds(base, BPW)], uidx)
    pltpu.sync_copy(iid_hbm.at[pl.ds(base, BPW)], iidx)
    pltpu.sync_copy(wb_hbm, wv)

    def split_body(k, _):
        off = pl.multiple_of(k * 16, 16)
        u = uidx[pl.ds(off, 16)]
        i = iidx[pl.ds(off, 16)]
        uwid[pl.ds(off, 16)] = lax.shift_right_logical(u, 2)
        iwid[pl.ds(off, 16)] = lax.shift_right_logical(i, 2)
        ublk[pl.ds(off, 16)] = lax.shift_left(u & 3, 5)
        iblk[pl.ds(off, 16)] = lax.shift_left(i & 3, 5)
        return 0

    lax.fori_loop(0, BPW // 16, split_body, 0)

    w_lo = wv[pl.ds(0, 16)]
    w_hi = wv[pl.ds(16, 16)]
    b0 = wv[pl.ds(32, 16)][0]
    lridx = lax.iota(jnp.int32, 16)

    for c in range(NCHUNK):
        cu = pltpu.async_copy(
            ut_hbm.at[uwid.at[pl.ds(c * CH, CH)]], uwide, sem)
        ci = pltpu.async_copy(
            it_hbm.at[iwid.at[pl.ds(c * CH, CH)]], iwide, sem)
        cu.wait()
        ci.wait()

        def group_body(g, _):
            goff = pl.multiple_of(c * CH + g * 16, 16)
            ridx = lridx + g * 16
            cu_base = ublk[pl.ds(goff, 16)]
            ci_base = iblk[pl.ds(goff, 16)]
            acc = jnp.zeros((16,), jnp.float32)
            for d in range(D):
                uc = plsc.load_gather(uwide, [ridx, cu_base + d])
                ic = plsc.load_gather(iwide, [ridx, ci_base + d])
                wd = w_lo[d] if d < 16 else w_hi[d - 16]
                acc = acc + uc * ic * wd
            v = 1.0 / (1.0 + jnp.exp(-(acc + b0)))
            outv[pl.ds(goff, 16)] = v
            return 0

        lax.fori_loop(0, CH // 16, group_body, 0)

    pltpu.sync_copy(outv, out_hbm.at[pl.ds(base, BPW)])


def kernel(user_ids, item_ids, user_table, item_table, W, b):
    wb = jnp.zeros((48,), jnp.float32)
    wb = wb.at[:D].set(W.reshape(D)).at[D].set(b[0])
    ut = user_table.reshape(-1, WIDE)
    it = item_table.reshape(-1, WIDE)
    return _gmf_sc(user_ids.astype(jnp.int32), item_ids.astype(jnp.int32),
                   ut, it, wb)


# speedup vs baseline: 3.0400x; 3.0400x over previous
import functools

import jax
import jax.numpy as jnp
from jax import lax
from jax.experimental import pallas as pl
from jax.experimental.pallas import tpu as pltpu
from jax.experimental.pallas import tpu_sc as plsc

BATCH = 16384
D = 32
STRIP = 128
NC = 2
NS = 16
NW = NC * NS
BPW = BATCH // NW
G = 16
NG = BPW // G
GF = 8

_mesh = plsc.VectorSubcoreMesh(core_axis_name="c", subcore_axis_name="s")


@functools.partial(
    pl.kernel,
    out_type=jax.ShapeDtypeStruct((BATCH,), jnp.float32),
    mesh=_mesh,
    scratch_types=[
        pltpu.VMEM((BPW,), jnp.int32),
        pltpu.VMEM((BPW,), jnp.int32),
        pltpu.VMEM((GF, D, STRIP), jnp.float32),
        pltpu.VMEM((G, D), jnp.float32),
        pltpu.VMEM((G, D), jnp.float32),
        pltpu.VMEM((48,), jnp.float32),
        pltpu.VMEM((BPW,), jnp.float32),
        pltpu.SemaphoreType.DMA,
    ],
    compiler_params=pltpu.CompilerParams(
        needs_layout_passes=False, use_tc_tiling_on_sc=True),
)
def _gmf_sc(uid_hbm, iid_hbm, ut_hbm, it_hbm, wb_hbm, out_hbm,
            uidx, iidx, strips, urows, irows, wv, outv, sem):
    wid = lax.axis_index("s") * NC + lax.axis_index("c")
    base = wid * BPW

    pltpu.sync_copy(uid_hbm.at[pl.ds(base, BPW)], uidx)
    pltpu.sync_copy(iid_hbm.at[pl.ds(base, BPW)], iidx)
    pltpu.sync_copy(wb_hbm, wv)

    lanes = lax.iota(jnp.int32, 16)
    d_lo = lanes
    d_hi = lanes + 16

    w_lo = wv[pl.ds(0, 16)]
    w_hi = wv[pl.ds(16, 16)]
    b0 = wv[pl.ds(32, 16)][0]

    def fetch_extract(tab_hbm, start, lane, rows_ref):
        for jbase in (0, GF):
            for j in range(GF):
                s = pl.multiple_of(start[jbase + j], 128)
                pltpu.async_copy(tab_hbm.at[:, pl.ds(s, STRIP)],
                                 strips.at[j], sem)
            for j in range(GF):
                pltpu.make_async_copy(tab_hbm.at[:, pl.ds(0, STRIP)],
                                      strips.at[j], sem).wait()
            for j in range(GF):
                jv = jnp.full((16,), j, jnp.int32)
                wl = jnp.full((16,), lane[jbase + j], jnp.int32)
                v_lo = plsc.load_gather(strips, [jv, d_lo, wl])
                v_hi = plsc.load_gather(strips, [jv, d_hi, wl])
                rows_ref[jbase + j, pl.ds(0, 16)] = v_lo
                rows_ref[jbase + j, pl.ds(16, 16)] = v_hi

    def group_body(g, _):
        off = pl.multiple_of(g * G, G)
        uids = uidx[pl.ds(off, 16)]
        iids = iidx[pl.ds(off, 16)]
        fetch_extract(ut_hbm, uids & ~127, uids & 127, urows)
        fetch_extract(it_hbm, iids & ~127, iids & 127, irows)
        acc = jnp.zeros((16,), jnp.float32)
        for j in range(G):
            s = jnp.sum(urows[j, pl.ds(0, 16)] * irows[j, pl.ds(0, 16)] * w_lo
                        + urows[j, pl.ds(16, 16)] * irows[j, pl.ds(16, 16)] * w_hi)
            acc = jnp.where(lanes == j, s, acc)
        outv[pl.ds(off, 16)] = 1.0 / (1.0 + jnp.exp(-(acc + b0)))
        return 0

    lax.fori_loop(0, NG, group_body, 0)

    pltpu.sync_copy(outv, out_hbm.at[pl.ds(base, BPW)])


def kernel(user_ids, item_ids, user_table, item_table, W, b):
    wb = jnp.zeros((48,), jnp.float32)
    wb = wb.at[:D].set(W.reshape(D)).at[D].set(b[0])
    return _gmf_sc(user_ids.astype(jnp.int32), item_ids.astype(jnp.int32),
                   user_table.T, item_table.T, wb)
